# baseline (speedup 1.0000x reference)
import jax
import jax.numpy as jnp
from jax import lax
from jax.experimental import pallas as pl
from jax.experimental.pallas import tpu as pltpu
from jax.experimental.pallas import tpu_sc as plsc

N = 50000
E = 800000
G = 128
H = 64
ALPHA = 0.01

N_PAD = 50176
E_PAD = 802816
HALF = N_PAD // 2
ACC_ROWS = HALF + 16
CHUNK = 128
MSTAGE = 3584
NCH_ACC = HALF // CHUNK
G_ACC = 256
R = 512
CAT_OFFS = (0, 7, 12, 16, 20, 22, 24, 28)
CAT_TOT = 32

_mesh = plsc.VectorSubcoreMesh(core_axis_name="c", subcore_axis_name="s")
_f32 = jnp.float32
_i32 = jnp.int32
_sc_params = pltpu.CompilerParams(needs_layout_passes=False,
                                  use_tc_tiling_on_sc=False)


NZCH = N_PAD // 512


CAP = 32256
BUFSZ = CAP + MSTAGE + 256 + 32
TRASH = CAP + MSTAGE + 256


def _bucket_body(src_hbm, dst_hbm, batch_hbm,
                 bsrc_hbm, bdst_hbm, bcnt_hbm, pd0_hbm, pd1_hbm,
                 pg0_hbm, pg1_hbm,
                 srcbuf, dstbuf, sstage, dstage, bstage, dchunk, bchunk,
                 ones, zbuf, cbuf, accd, accg, sem):
    cid = lax.axis_index("c")
    sid = lax.axis_index("s")
    lo = cid * HALF
    ebase = sid * (E_PAD // 16)

    def zb(i, _):
        zbuf[pl.ds(i * 16, 16)] = jnp.zeros((16,), _f32)
        return 0
    lax.fori_loop(0, 32, zb, 0)
    for j in range(CHUNK // 16):
        ones[pl.ds(j * 16, 16)] = jnp.ones((16,), _f32)

    def za(k, _):
        c = sid + k * 16
        @pl.when(c < NZCH)
        def _():
            pltpu.sync_copy(zbuf, accd.at[pl.ds(c * 512, 512)])
        return 0
    lax.fori_loop(0, 7, za, 0)

    @pl.when(sid == 0)
    def _():
        pltpu.sync_copy(zbuf.at[pl.ds(0, G_ACC)], accg)
    plsc.subcore_barrier()

    def outer(c, ptr):
        pltpu.sync_copy(src_hbm.at[pl.ds(ebase + c * MSTAGE, MSTAGE)], sstage)
        pltpu.sync_copy(dst_hbm.at[pl.ds(ebase + c * MSTAGE, MSTAGE)], dstage)

        def deg_chunk(i, _):
            def cp(j, _):
                dchunk[pl.ds(j * 16, 16)] = dstage[
                    pl.ds(cid * (MSTAGE // 2) + i * CHUNK + j * 16, 16)]
                return 0
            lax.fori_loop(0, CHUNK // 16, cp, 0)
            pltpu.sync_copy(ones, accd.at[dchunk], add=True)
            return 0
        lax.fori_loop(0, (MSTAGE // 2) // CHUNK, deg_chunk, 0)

        def inner(i, ptr):
            s16 = sstage[pl.ds(i * 16, 16)]
            d16 = dstage[pl.ds(i * 16, 16)]
            m = (d16 >= lo) & (d16 < lo + HALF)
            mi = m.astype(_i32)
            cum = plsc.cumsum(mi)
            lane = lax.iota(_i32, 16)
            pos = jnp.where(m, ptr + cum - mi, TRASH + lane)
            plsc.store_scatter(srcbuf, [pos], s16)
            plsc.store_scatter(dstbuf, [pos], d16 - lo)
            return ptr + jnp.sum(mi)
        return lax.fori_loop(0, MSTAGE // 16, inner, ptr)
    kcnt = lax.fori_loop(0, (E_PAD // 16) // MSTAGE, outer,
                         jnp.asarray(0, _i32))

    def fill(j, _):
        srcbuf[pl.ds(kcnt + j * 16, 16)] = jnp.zeros((16,), _i32)
        dstbuf[pl.ds(kcnt + j * 16, 16)] = HALF + lax.iota(_i32, 16)
        return 0
    lax.fori_loop(0, (MSTAGE + 256) // 16, fill, 0)

    base = (cid * 16 + sid) * CAP
    nblk = (kcnt + 256 + MSTAGE - 1) // MSTAGE

    def wr(c, _):
        pltpu.sync_copy(srcbuf.at[pl.ds(c * MSTAGE, MSTAGE)],
                        bsrc_hbm.at[pl.ds(base + c * MSTAGE, MSTAGE)])
        pltpu.sync_copy(dstbuf.at[pl.ds(c * MSTAGE, MSTAGE)],
                        bdst_hbm.at[pl.ds(base + c * MSTAGE, MSTAGE)])
        return 0
    lax.fori_loop(0, nblk, wr, 0)

    nch = ((kcnt + 2 * CHUNK - 1) // (2 * CHUNK)) * 2
    cbuf[pl.ds(0, 16)] = jnp.zeros((16,), _i32) + nch
    pltpu.sync_copy(cbuf, bcnt_hbm.at[pl.ds((cid * 16 + sid) * 128, 128)])

    wid = sid * 2 + cid
    def g_outer(k, _):
        c = wid + k * 32
        @pl.when(c < NZCH)
        def _():
            pltpu.sync_copy(batch_hbm.at[pl.ds(c * 512, 512)], bstage)
            def g_chunk(i, _):
                bchunk[pl.ds(0, 16)] = bstage[pl.ds(i * 16, 16)]
                pltpu.sync_copy(ones.at[pl.ds(0, 16)], accg.at[bchunk],
                                add=True)
                return 0
            lax.fori_loop(0, 32, g_chunk, 0)
        return 0
    lax.fori_loop(0, 4, g_outer, 0)

    plsc.subcore_barrier()
    def co(k, _):
        c = sid + k * 16
        @pl.when((c < NZCH) & (cid == 0))
        def _():
            pltpu.sync_copy(accd.at[pl.ds(c * 512, 512)],
                            pd0_hbm.at[pl.ds(c * 512, 512)])
        @pl.when((c < NZCH) & (cid == 1))
        def _():
            pltpu.sync_copy(accd.at[pl.ds(c * 512, 512)],
                            pd1_hbm.at[pl.ds(c * 512, 512)])
        return 0
    lax.fori_loop(0, 7, co, 0)

    @pl.when((sid == 0) & (cid == 0))
    def _():
        pltpu.sync_copy(accg, pg0_hbm)

    @pl.when((sid == 0) & (cid == 1))
    def _():
        pltpu.sync_copy(accg, pg1_hbm)


def _bucket(src_p, dst_p, batch_p):
    k = pl.kernel(
        _bucket_body,
        out_type=(jax.ShapeDtypeStruct((32 * CAP,), _i32),
                  jax.ShapeDtypeStruct((32 * CAP,), _i32),
                  jax.ShapeDtypeStruct((32 * 128,), _i32),
                  jax.ShapeDtypeStruct((N_PAD,), _f32),
                  jax.ShapeDtypeStruct((N_PAD,), _f32),
                  jax.ShapeDtypeStruct((G_ACC,), _f32),
                  jax.ShapeDtypeStruct((G_ACC,), _f32)),
        mesh=_mesh,
        scratch_types=[
            pltpu.VMEM((BUFSZ,), _i32),
            pltpu.VMEM((BUFSZ,), _i32),
            pltpu.VMEM((MSTAGE,), _i32),
            pltpu.VMEM((MSTAGE,), _i32),
            pltpu.VMEM((512,), _i32),
            pltpu.VMEM((CHUNK,), _i32),
            pltpu.VMEM((16,), _i32),
            pltpu.VMEM((CHUNK,), _f32),
            pltpu.VMEM((512,), _f32),
            pltpu.VMEM((128,), _i32),
            pltpu.VMEM_SHARED((N_PAD,), _f32),
            pltpu.VMEM_SHARED((G_ACC,), _f32),
            pltpu.SemaphoreType.DMA,
        ],
        compiler_params=_sc_params,
    )
    return k(src_p, dst_p, batch_p)


def _msgpass_body(s_hbm, bsrc_hbm, bdst_hbm, bcnt_hbm, out_hbm,
                  rows0, rows1, sblk, dblk, dch0, dch1, cbuf, acc,
                  semi, semg, semg1, sems):
    cid = lax.axis_index("c")
    sid = lax.axis_index("s")
    lo = cid * HALF

    def zr(i, _):
        rows0[i // 4, pl.ds((i % 4) * 16, 16)] = jnp.zeros((16,), _f32)
        return 0
    lax.fori_loop(0, CHUNK * 4, zr, 0)

    def za(k, _):
        c = sid + k * 16
        @pl.when(c < NCH_ACC)
        def _():
            pltpu.sync_copy(rows0, acc.at[pl.ds(c * CHUNK, CHUNK)])
        return 0
    lax.fori_loop(0, 13, za, 0)
    plsc.subcore_barrier()

    pltpu.sync_copy(bcnt_hbm.at[pl.ds((cid * 16 + sid) * 128, 128)], cbuf)
    nch = cbuf[pl.ds(0, 16)][0]
    base = (cid * 16 + sid) * CAP

    def chunk_body(c2, _):
        c0 = 2 * c2
        i0s = pltpu.async_copy(bsrc_hbm.at[pl.ds(base + c0 * CHUNK, CHUNK)],
                               sblk, semi)
        i0d = pltpu.async_copy(bdst_hbm.at[pl.ds(base + c0 * CHUNK, CHUNK)],
                               dch0, semi)
        i1s = pltpu.async_copy(
            bsrc_hbm.at[pl.ds(base + (c0 + 1) * CHUNK, CHUNK)], dblk, semi)
        i1d = pltpu.async_copy(
            bdst_hbm.at[pl.ds(base + (c0 + 1) * CHUNK, CHUNK)], dch1, semi)
        i0s.wait()
        i0d.wait()
        i1s.wait()
        i1d.wait()
        g0 = pltpu.async_copy(s_hbm.at[sblk], rows0, semg)
        g1 = pltpu.async_copy(s_hbm.at[dblk], rows1, semg1)
        g0.wait()
        s0 = pltpu.async_copy(rows0, acc.at[dch0], sems, add=True)
        g1.wait()
        s1 = pltpu.async_copy(rows1, acc.at[dch1], sems, add=True)
        s0.wait()
        s1.wait()
        return 0
    lax.fori_loop(0, nch // 2, chunk_body, 0)

    plsc.subcore_barrier()
    def co(k, _):
        c = sid + k * 16
        @pl.when(c < NCH_ACC)
        def _():
            pltpu.sync_copy(acc.at[pl.ds(c * CHUNK, CHUNK)],
                            out_hbm.at[pl.ds(lo + c * CHUNK, CHUNK)])
        return 0
    lax.fori_loop(0, 13, co, 0)


def _msgpass(s, bsrc, bdst, bcnt):
    k = pl.kernel(
        _msgpass_body,
        out_type=jax.ShapeDtypeStruct((N_PAD, H), _f32),
        mesh=_mesh,
        scratch_types=[
            pltpu.VMEM((CHUNK, H), _f32),
            pltpu.VMEM((CHUNK, H), _f32),
            pltpu.VMEM((CHUNK,), _i32),
            pltpu.VMEM((CHUNK,), _i32),
            pltpu.VMEM((CHUNK,), _i32),
            pltpu.VMEM((CHUNK,), _i32),
            pltpu.VMEM((128,), _i32),
            pltpu.VMEM_SHARED((ACC_ROWS, H), _f32),
            pltpu.SemaphoreType.DMA,
            pltpu.SemaphoreType.DMA,
            pltpu.SemaphoreType.DMA,
            pltpu.SemaphoreType.DMA,
        ],
        compiler_params=_sc_params,
    )
    return k(s, bsrc, bdst, bcnt)


def _pool_body(x4, x1, x2, x3, pg0_hbm, pg1_hbm, o4, o1, o2, o3,
               pgv0, pgv1, st_s, cn_s, buf, obuf, sem):
    cid = lax.axis_index("c")
    sid = lax.axis_index("s")
    wid = sid * 2 + cid
    pltpu.sync_copy(pg0_hbm, pgv0)
    pltpu.sync_copy(pg1_hbm, pgv1)

    def cs(i, run):
        c16 = pgv0[pl.ds(i * 16, 16)] + pgv1[pl.ds(i * 16, 16)]
        inc = plsc.cumsum(c16)
        st_s[pl.ds(i * 16, 16)] = (run + inc - c16).astype(_i32)
        cn_s[pl.ds(i * 16, 16)] = c16.astype(_i32)
        return run + jnp.sum(c16)
    lax.fori_loop(0, G // 16, cs, jnp.asarray(0.0, _f32))

    for xin, oout in ((x4, o4), (x1, o1), (x2, o2), (x3, o3)):
        def per_graph(gi, _, xin=xin):
            g = wid * 4 + gi
            start = st_s[pl.ds(g, 16)][0]
            end = start + cn_s[pl.ds(g, 16)][0]
            abase = (start // 8) * 8

            def ch_body(c, accs):
                pltpu.sync_copy(xin.at[pl.ds(abase + c * 32, 32)], buf)

                def row(rr, accs):
                    a0, a1, a2, a3 = accs
                    r = abase + c * 32 + rr
                    pred = (r >= start) & (r < end)
                    v0 = buf[rr, pl.ds(0, 16)]
                    v1 = buf[rr, pl.ds(16, 16)]
                    v2 = buf[rr, pl.ds(32, 16)]
                    v3 = buf[rr, pl.ds(48, 16)]
                    a0 = jnp.where(pred, jnp.maximum(a0, v0), a0)
                    a1 = jnp.where(pred, jnp.maximum(a1, v1), a1)
                    a2 = jnp.where(pred, jnp.maximum(a2, v2), a2)
                    a3 = jnp.where(pred, jnp.maximum(a3, v3), a3)
                    return (a0, a1, a2, a3)
                return lax.fori_loop(0, 32, row, accs)

            neg = jnp.full((16,), -jnp.inf, _f32)
            nch = (end - abase + 31) // 32
            a0, a1, a2, a3 = lax.fori_loop(0, nch, ch_body,
                                           (neg, neg, neg, neg))
            obuf[gi, pl.ds(0, 16)] = a0
            obuf[gi, pl.ds(16, 16)] = a1
            obuf[gi, pl.ds(32, 16)] = a2
            obuf[gi, pl.ds(48, 16)] = a3
            return 0
        lax.fori_loop(0, 4, per_graph, 0)
        pltpu.sync_copy(obuf, oout.at[wid])


def _pool(x4, x1, x2, x3, pg0, pg1):
    k = pl.kernel(
        _pool_body,
        out_type=tuple(jax.ShapeDtypeStruct((32, 4, H), _f32)
                       for _ in range(4)),
        mesh=_mesh,
        scratch_types=[
            pltpu.VMEM((G_ACC,), _f32),
            pltpu.VMEM((G_ACC,), _f32),
            pltpu.VMEM((G + 16,), _i32),
            pltpu.VMEM((G + 16,), _i32),
            pltpu.VMEM((32, H), _f32),
            pltpu.VMEM((4, H), _f32),
            pltpu.SemaphoreType.DMA,
        ],
        compiler_params=_sc_params,
    )
    return k(x4, x1, x2, x3, pg0, pg1)


def _stage0_kernel(x_ref, p0_ref, p1_ref, wcat_ref, wc_ref, b_ref,
                   s0_ref, dis_ref):
    x = x_ref[...]
    iota = lax.broadcasted_iota(_i32, (R, CAT_TOT), 1)
    oh = jnp.zeros((R, CAT_TOT), _f32)
    for i, off in enumerate(CAT_OFFS):
        idx = x[:, i:i + 1].astype(_i32) + off
        oh = oh + (iota == idx).astype(_f32)
    lin = (jnp.dot(oh, wcat_ref[...], preferred_element_type=_f32)
           + jnp.dot(x[:, 8:16], wc_ref[...], preferred_element_type=_f32)
           + b_ref[...])
    dis = lax.rsqrt(1.0 + p0_ref[...] + p1_ref[...])
    dis_ref[...] = dis
    s0_ref[...] = lin * dis


def _stage0(x_p, p0, p1, wcat, wc, b):
    return pl.pallas_call(
        _stage0_kernel,
        grid=(N_PAD // R,),
        in_specs=[pl.BlockSpec((R, 16), lambda i: (i, 0)),
                  pl.BlockSpec((R, 1), lambda i: (i, 0)),
                  pl.BlockSpec((R, 1), lambda i: (i, 0)),
                  pl.BlockSpec((CAT_TOT, H), lambda i: (0, 0)),
                  pl.BlockSpec((8, H), lambda i: (0, 0)),
                  pl.BlockSpec((1, H), lambda i: (0, 0))],
        out_specs=[pl.BlockSpec((R, H), lambda i: (i, 0)),
                   pl.BlockSpec((R, 1), lambda i: (i, 0))],
        out_shape=[jax.ShapeDtypeStruct((N_PAD, H), _f32),
                   jax.ShapeDtypeStruct((N_PAD, 1), _f32)],
    )(x_p, p0, p1, wcat, wc, b)


def _stage_mid_kernel(agg_ref, s_ref, dis_ref, w_ref, b_ref, h_ref, sn_ref):
    d = dis_ref[...]
    hh = d * (agg_ref[...] + s_ref[...])
    hh = jnp.where(hh > 0, hh, ALPHA * hh)
    h_ref[...] = hh
    sn_ref[...] = (jnp.dot(hh, w_ref[...], preferred_element_type=_f32)
                   + b_ref[...]) * d


def _stage_mid(agg, s, dis, w, b):
    return pl.pallas_call(
        _stage_mid_kernel,
        grid=(N_PAD // R,),
        in_specs=[pl.BlockSpec((R, H), lambda i: (i, 0)),
                  pl.BlockSpec((R, H), lambda i: (i, 0)),
                  pl.BlockSpec((R, 1), lambda i: (i, 0)),
                  pl.BlockSpec((H, H), lambda i: (0, 0)),
                  pl.BlockSpec((1, H), lambda i: (0, 0))],
        out_specs=[pl.BlockSpec((R, H), lambda i: (i, 0)),
                   pl.BlockSpec((R, H), lambda i: (i, 0))],
        out_shape=[jax.ShapeDtypeStruct((N_PAD, H), _f32),
                   jax.ShapeDtypeStruct((N_PAD, H), _f32)],
    )(agg, s, dis, w, b)


def _stage3_kernel(agg_ref, s_ref, dis_ref, h0_ref, w_ref, b_ref,
                   x2_ref, x3_ref, sn_ref):
    d = dis_ref[...]
    hh = d * (agg_ref[...] + s_ref[...])
    hh = jnp.where(hh > 0, hh, ALPHA * hh)
    x2_ref[...] = hh
    x3 = h0_ref[...] + hh
    x3_ref[...] = x3
    sn_ref[...] = (jnp.dot(x3, w_ref[...], preferred_element_type=_f32)
                   + b_ref[...]) * d


def _stage3(agg, s, dis, h0, w, b):
    return pl.pallas_call(
        _stage3_kernel,
        grid=(N_PAD // R,),
        in_specs=[pl.BlockSpec((R, H), lambda i: (i, 0)),
                  pl.BlockSpec((R, H), lambda i: (i, 0)),
                  pl.BlockSpec((R, 1), lambda i: (i, 0)),
                  pl.BlockSpec((R, H), lambda i: (i, 0)),
                  pl.BlockSpec((H, H), lambda i: (0, 0)),
                  pl.BlockSpec((1, H), lambda i: (0, 0))],
        out_specs=[pl.BlockSpec((R, H), lambda i: (i, 0)),
                   pl.BlockSpec((R, H), lambda i: (i, 0)),
                   pl.BlockSpec((R, H), lambda i: (i, 0))],
        out_shape=[jax.ShapeDtypeStruct((N_PAD, H), _f32)] * 3,
    )(agg, s, dis, h0, w, b)


def _stage4_kernel(agg_ref, s_ref, dis_ref, x4_ref):
    hh = dis_ref[...] * (agg_ref[...] + s_ref[...])
    x4_ref[...] = jnp.where(hh > 0, hh, ALPHA * hh)


def _stage4(agg, s, dis):
    return pl.pallas_call(
        _stage4_kernel,
        grid=(N_PAD // R,),
        in_specs=[pl.BlockSpec((R, H), lambda i: (i, 0)),
                  pl.BlockSpec((R, H), lambda i: (i, 0)),
                  pl.BlockSpec((R, 1), lambda i: (i, 0))],
        out_specs=pl.BlockSpec((R, H), lambda i: (i, 0)),
        out_shape=jax.ShapeDtypeStruct((N_PAD, H), _f32),
    )(agg, s, dis)


def _mlp_kernel(p4_ref, p1_ref, p2_ref, p3_ref, wa_ref, wb_ref, wc_ref,
                wd_ref, b1_ref, gm_ref, bt_ref, w2_ref, b2_ref, o_ref):
    y = (jnp.dot(p4_ref[...], wa_ref[...], preferred_element_type=_f32)
         + jnp.dot(p1_ref[...], wb_ref[...], preferred_element_type=_f32)
         + jnp.dot(p2_ref[...], wc_ref[...], preferred_element_type=_f32)
         + jnp.dot(p3_ref[...], wd_ref[...], preferred_element_type=_f32)
         + b1_ref[...])
    mu = jnp.mean(y, axis=0, keepdims=True)
    var = jnp.mean((y - mu) * (y - mu), axis=0, keepdims=True)
    yn = (y - mu) * lax.rsqrt(var + 1e-5) * gm_ref[...] + bt_ref[...]
    yr = jnp.maximum(yn, 0.0)
    o_ref[...] = jnp.dot(yr, w2_ref[...], preferred_element_type=_f32) + b2_ref[...]


def _mlp(p4, p1, p2, p3, wa, wb, wc, wd, b1, gm, bt, w2, b2):
    return pl.pallas_call(
        _mlp_kernel,
        out_shape=jax.ShapeDtypeStruct((G, 1), _f32),
    )(p4, p1, p2, p3, wa, wb, wc, wd, b1, gm, bt, w2, b2)


def kernel(x, edge_index, batch, params):
    p = params
    src = edge_index[0].astype(_i32)
    dst = edge_index[1].astype(_i32)
    pad_e = E_PAD - E
    src_p = jnp.concatenate([src, jnp.zeros((pad_e,), _i32)])
    dst_p = jnp.concatenate([dst, jnp.full((pad_e,), N_PAD - 1, _i32)])
    batch_p = jnp.concatenate([batch.astype(_i32),
                               jnp.full((N_PAD - N,), G_ACC - 1, _i32)])
    x_p = jnp.pad(x, ((0, N_PAD - N), (0, 0)))

    bsrc, bdst, bcnt, pd0, pd1, pg0, pg1 = _bucket(src_p, dst_p, batch_p)
    p0 = pd0.reshape(N_PAD, 1)
    p1 = pd1.reshape(N_PAD, 1)

    wcat = jnp.pad(p['W_init'][:31], ((0, 1), (0, 0)))
    wc = p['W_init'][31:]
    s0, dis = _stage0(x_p, p0, p1, wcat, wc, p['b_init'].reshape(1, H))

    agg0 = _msgpass(s0, bsrc, bdst, bcnt)
    h0, s1 = _stage_mid(agg0, s0, dis, p['W_head'], p['b_head'].reshape(1, H))
    agg1 = _msgpass(s1, bsrc, bdst, bcnt)
    x1, s2 = _stage_mid(agg1, s1, dis, p['W_body'], p['b_body'].reshape(1, H))
    agg2 = _msgpass(s2, bsrc, bdst, bcnt)
    x2, x3, s3 = _stage3(agg2, s2, dis, h0, p['W_tail'],
                         p['b_tail'].reshape(1, H))
    agg3 = _msgpass(s3, bsrc, bdst, bcnt)
    x4 = _stage4(agg3, s3, dis)

    p4, p1m, p2m, p3m = _pool(x4, x1, x2, x3, pg0, pg1)
    p4, p1m, p2m, p3m = (a.reshape(G, H) for a in (p4, p1m, p2m, p3m))

    w1 = p['W_mlp1']
    out = _mlp(p4, p1m, p2m, p3m,
               w1[0:64], w1[64:128], w1[128:192], w1[192:256],
               p['b_mlp1'].reshape(1, H),
               p['bn_gamma'].reshape(1, H), p['bn_beta'].reshape(1, H),
               p['W_mlp2'], p['b_mlp2'].reshape(1, 1))
    return out.reshape(G)


# speedup vs baseline: 1.1985x; 1.1985x over previous
import jax
import jax.numpy as jnp
from jax import lax
from jax.experimental import pallas as pl
from jax.experimental.pallas import tpu as pltpu
from jax.experimental.pallas import tpu_sc as plsc

N = 50000
E = 800000
G = 128
H = 64
ALPHA = 0.01

N_PAD = 50176
E_PAD = 802816
HALF = N_PAD // 2
ACC_ROWS = HALF + 16
CHUNK = 128
MSTAGE = 3584
NCH_ACC = HALF // CHUNK
G_ACC = 256
R = 512
CAT_OFFS = (0, 7, 12, 16, 20, 22, 24, 28)
CAT_TOT = 32

_mesh = plsc.VectorSubcoreMesh(core_axis_name="c", subcore_axis_name="s")
_f32 = jnp.float32
_i32 = jnp.int32
_sc_params = pltpu.CompilerParams(needs_layout_passes=False,
                                  use_tc_tiling_on_sc=False)


NZCH = N_PAD // 512


def _stats_body(dst_hbm, batch_hbm, pd0_hbm, pd1_hbm, pg0_hbm, pg1_hbm,
                estage, bstage, dchunk, bchunk, ones, zbuf, accd, accg, sem):
    cid = lax.axis_index("c")
    sid = lax.axis_index("s")
    wid = sid * 2 + cid

    def zb(i, _):
        zbuf[pl.ds(i * 16, 16)] = jnp.zeros((16,), _f32)
        return 0
    lax.fori_loop(0, 32, zb, 0)
    for j in range(CHUNK // 16):
        ones[pl.ds(j * 16, 16)] = jnp.ones((16,), _f32)

    def za(k, _):
        c = sid + k * 16
        @pl.when(c < NZCH)
        def _():
            pltpu.sync_copy(zbuf, accd.at[pl.ds(c * 512, 512)])
        return 0
    lax.fori_loop(0, 7, za, 0)

    @pl.when(sid == 0)
    def _():
        pltpu.sync_copy(zbuf.at[pl.ds(0, G_ACC)], accg)
    plsc.subcore_barrier()

    pltpu.sync_copy(dst_hbm.at[pl.ds(wid * (E_PAD // 32), E_PAD // 32)], estage)

    def deg_chunk(i, _):
        def cp(j, _):
            dchunk[pl.ds(j * 16, 16)] = estage[pl.ds(i * CHUNK + j * 16, 16)]
            return 0
        lax.fori_loop(0, CHUNK // 16, cp, 0)
        pltpu.sync_copy(ones, accd.at[dchunk], add=True)
        return 0
    lax.fori_loop(0, (E_PAD // 32) // CHUNK, deg_chunk, 0)

    def g_outer(k, _):
        c = wid + k * 32
        @pl.when(c < NZCH)
        def _():
            pltpu.sync_copy(batch_hbm.at[pl.ds(c * 512, 512)], bstage)
            def g_chunk(i, _):
                bchunk[pl.ds(0, 16)] = bstage[pl.ds(i * 16, 16)]
                pltpu.sync_copy(ones.at[pl.ds(0, 16)], accg.at[bchunk],
                                add=True)
                return 0
            lax.fori_loop(0, 32, g_chunk, 0)
        return 0
    lax.fori_loop(0, 4, g_outer, 0)

    plsc.subcore_barrier()
    def co(k, _):
        c = sid + k * 16
        @pl.when((c < NZCH) & (cid == 0))
        def _():
            pltpu.sync_copy(accd.at[pl.ds(c * 512, 512)],
                            pd0_hbm.at[pl.ds(c * 512, 512)])
        @pl.when((c < NZCH) & (cid == 1))
        def _():
            pltpu.sync_copy(accd.at[pl.ds(c * 512, 512)],
                            pd1_hbm.at[pl.ds(c * 512, 512)])
        return 0
    lax.fori_loop(0, 7, co, 0)

    @pl.when((sid == 0) & (cid == 0))
    def _():
        pltpu.sync_copy(accg, pg0_hbm)

    @pl.when((sid == 0) & (cid == 1))
    def _():
        pltpu.sync_copy(accg, pg1_hbm)


def _stats(dst_deg, batch_p):
    k = pl.kernel(
        _stats_body,
        out_type=(jax.ShapeDtypeStruct((N_PAD,), _f32),
                  jax.ShapeDtypeStruct((N_PAD,), _f32),
                  jax.ShapeDtypeStruct((G_ACC,), _f32),
                  jax.ShapeDtypeStruct((G_ACC,), _f32)),
        mesh=_mesh,
        scratch_types=[
            pltpu.VMEM((E_PAD // 32,), _i32),
            pltpu.VMEM((512,), _i32),
            pltpu.VMEM((CHUNK,), _i32),
            pltpu.VMEM((16,), _i32),
            pltpu.VMEM((CHUNK,), _f32),
            pltpu.VMEM((512,), _f32),
            pltpu.VMEM_SHARED((N_PAD,), _f32),
            pltpu.VMEM_SHARED((G_ACC,), _f32),
            pltpu.SemaphoreType.DMA,
        ],
    )
    return k(dst_deg, batch_p)


CAP = 32256
BLK = 16 * CHUNK
BUFSZ = CAP + MSTAGE + BLK + 32
TRASH = CAP + MSTAGE + BLK


def _bucket_body(src_hbm, dst_hbm, bsrc_hbm, bdst_hbm, bcnt_hbm,
                 srcbuf, dstbuf, sstage, dstage, cbuf, sem):
    cid = lax.axis_index("c")
    sid = lax.axis_index("s")
    lo = cid * HALF
    ebase = sid * (E_PAD // 16)

    def outer(c, ptr):
        pltpu.sync_copy(src_hbm.at[pl.ds(ebase + c * MSTAGE, MSTAGE)], sstage)
        pltpu.sync_copy(dst_hbm.at[pl.ds(ebase + c * MSTAGE, MSTAGE)], dstage)

        def inner(i, ptr):
            s16 = sstage[pl.ds(i * 16, 16)]
            d16 = dstage[pl.ds(i * 16, 16)]
            m = (d16 >= lo) & (d16 < lo + HALF)
            mi = m.astype(_i32)
            cum = plsc.cumsum(mi)
            lane = lax.iota(_i32, 16)
            pos = jnp.where(m, ptr + cum - mi, TRASH + lane)
            plsc.store_scatter(srcbuf, [pos], s16)
            plsc.store_scatter(dstbuf, [pos], d16 - lo)
            return ptr + jnp.sum(mi)
        return lax.fori_loop(0, MSTAGE // 16, inner, ptr)
    kcnt = lax.fori_loop(0, (E_PAD // 16) // MSTAGE, outer,
                         jnp.asarray(0, _i32))

    def fill(j, _):
        srcbuf[pl.ds(kcnt + j * 16, 16)] = jnp.zeros((16,), _i32)
        dstbuf[pl.ds(kcnt + j * 16, 16)] = HALF + lax.iota(_i32, 16)
        return 0
    lax.fori_loop(0, (MSTAGE + 256) // 16, fill, 0)

    base = (cid * 16 + sid) * CAP
    nblk = (kcnt + 256 + MSTAGE - 1) // MSTAGE

    def wr(c, _):
        pltpu.sync_copy(srcbuf.at[pl.ds(c * MSTAGE, MSTAGE)],
                        bsrc_hbm.at[pl.ds(base + c * MSTAGE, MSTAGE)])
        pltpu.sync_copy(dstbuf.at[pl.ds(c * MSTAGE, MSTAGE)],
                        bdst_hbm.at[pl.ds(base + c * MSTAGE, MSTAGE)])
        return 0
    lax.fori_loop(0, nblk, wr, 0)

    nch = ((kcnt + 2 * CHUNK - 1) // (2 * CHUNK)) * 2
    cbuf[pl.ds(0, 16)] = jnp.zeros((16,), _i32) + nch
    pltpu.sync_copy(cbuf, bcnt_hbm.at[pl.ds((cid * 16 + sid) * 128, 128)])


def _bucket(src_p, dst_p):
    k = pl.kernel(
        _bucket_body,
        out_type=(jax.ShapeDtypeStruct((32 * CAP,), _i32),
                  jax.ShapeDtypeStruct((32 * CAP,), _i32),
                  jax.ShapeDtypeStruct((32 * 128,), _i32)),
        mesh=_mesh,
        scratch_types=[
            pltpu.VMEM((BUFSZ,), _i32),
            pltpu.VMEM((BUFSZ,), _i32),
            pltpu.VMEM((MSTAGE,), _i32),
            pltpu.VMEM((MSTAGE,), _i32),
            pltpu.VMEM((128,), _i32),
            pltpu.SemaphoreType.DMA,
        ],
        compiler_params=_sc_params,
    )
    return k(src_p, dst_p)


def _msgpass_body(s_hbm, bsrc_hbm, bdst_hbm, bcnt_hbm, out_hbm,
                  rows0, rows1, sblk, dblk, dch0, dch1, cbuf, acc,
                  semi, semg, semg1, sems):
    cid = lax.axis_index("c")
    sid = lax.axis_index("s")
    lo = cid * HALF

    def zr(i, _):
        rows0[i // 4, pl.ds((i % 4) * 16, 16)] = jnp.zeros((16,), _f32)
        return 0
    lax.fori_loop(0, CHUNK * 4, zr, 0)

    def za(k, _):
        c = sid + k * 16
        @pl.when(c < NCH_ACC)
        def _():
            pltpu.sync_copy(rows0, acc.at[pl.ds(c * CHUNK, CHUNK)])
        return 0
    lax.fori_loop(0, 13, za, 0)
    plsc.subcore_barrier()

    pltpu.sync_copy(bcnt_hbm.at[pl.ds((cid * 16 + sid) * 128, 128)], cbuf)
    nch = cbuf[pl.ds(0, 16)][0]
    base = (cid * 16 + sid) * CAP

    def chunk_body(c2, _):
        c0 = 2 * c2
        i0s = pltpu.async_copy(bsrc_hbm.at[pl.ds(base + c0 * CHUNK, CHUNK)],
                               sblk, semi)
        i0d = pltpu.async_copy(bdst_hbm.at[pl.ds(base + c0 * CHUNK, CHUNK)],
                               dch0, semi)
        i1s = pltpu.async_copy(
            bsrc_hbm.at[pl.ds(base + (c0 + 1) * CHUNK, CHUNK)], dblk, semi)
        i1d = pltpu.async_copy(
            bdst_hbm.at[pl.ds(base + (c0 + 1) * CHUNK, CHUNK)], dch1, semi)
        i0s.wait()
        i0d.wait()
        i1s.wait()
        i1d.wait()
        g0 = pltpu.async_copy(s_hbm.at[sblk], rows0, semg)
        g1 = pltpu.async_copy(s_hbm.at[dblk], rows1, semg1)
        g0.wait()
        s0 = pltpu.async_copy(rows0, acc.at[dch0], sems, add=True)
        g1.wait()
        s1 = pltpu.async_copy(rows1, acc.at[dch1], sems, add=True)
        s0.wait()
        s1.wait()
        return 0
    lax.fori_loop(0, nch // 2, chunk_body, 0)

    plsc.subcore_barrier()
    def co(k, _):
        c = sid + k * 16
        @pl.when(c < NCH_ACC)
        def _():
            pltpu.sync_copy(acc.at[pl.ds(c * CHUNK, CHUNK)],
                            out_hbm.at[pl.ds(lo + c * CHUNK, CHUNK)])
        return 0
    lax.fori_loop(0, 13, co, 0)


def _msgpass(s, bsrc, bdst, bcnt):
    k = pl.kernel(
        _msgpass_body,
        out_type=jax.ShapeDtypeStruct((N_PAD, H), _f32),
        mesh=_mesh,
        scratch_types=[
            pltpu.VMEM((CHUNK, H), _f32),
            pltpu.VMEM((CHUNK, H), _f32),
            pltpu.VMEM((CHUNK,), _i32),
            pltpu.VMEM((CHUNK,), _i32),
            pltpu.VMEM((CHUNK,), _i32),
            pltpu.VMEM((CHUNK,), _i32),
            pltpu.VMEM((128,), _i32),
            pltpu.VMEM_SHARED((ACC_ROWS, H), _f32),
            pltpu.SemaphoreType.DMA,
            pltpu.SemaphoreType.DMA,
            pltpu.SemaphoreType.DMA,
            pltpu.SemaphoreType.DMA,
        ],
        compiler_params=_sc_params,
    )
    return k(s, bsrc, bdst, bcnt)


def _pool_body(x4, x1, x2, x3, pg0_hbm, pg1_hbm, o4, o1, o2, o3,
               pgv0, pgv1, st_s, cn_s, buf, obuf, sem):
    cid = lax.axis_index("c")
    sid = lax.axis_index("s")
    wid = sid * 2 + cid
    pltpu.sync_copy(pg0_hbm, pgv0)
    pltpu.sync_copy(pg1_hbm, pgv1)

    def cs(i, run):
        c16 = pgv0[pl.ds(i * 16, 16)] + pgv1[pl.ds(i * 16, 16)]
        inc = plsc.cumsum(c16)
        st_s[pl.ds(i * 16, 16)] = (run + inc - c16).astype(_i32)
        cn_s[pl.ds(i * 16, 16)] = c16.astype(_i32)
        return run + jnp.sum(c16)
    lax.fori_loop(0, G // 16, cs, jnp.asarray(0.0, _f32))

    for xin, oout in ((x4, o4), (x1, o1), (x2, o2), (x3, o3)):
        def per_graph(gi, _, xin=xin):
            g = wid * 4 + gi
            start = st_s[pl.ds(g, 16)][0]
            end = start + cn_s[pl.ds(g, 16)][0]
            abase = (start // 8) * 8

            def ch_body(c, accs):
                pltpu.sync_copy(xin.at[pl.ds(abase + c * 32, 32)], buf)

                def row(rr, accs):
                    a0, a1, a2, a3 = accs
                    r = abase + c * 32 + rr
                    pred = (r >= start) & (r < end)
                    v0 = buf[rr, pl.ds(0, 16)]
                    v1 = buf[rr, pl.ds(16, 16)]
                    v2 = buf[rr, pl.ds(32, 16)]
                    v3 = buf[rr, pl.ds(48, 16)]
                    a0 = jnp.where(pred, jnp.maximum(a0, v0), a0)
                    a1 = jnp.where(pred, jnp.maximum(a1, v1), a1)
                    a2 = jnp.where(pred, jnp.maximum(a2, v2), a2)
                    a3 = jnp.where(pred, jnp.maximum(a3, v3), a3)
                    return (a0, a1, a2, a3)
                return lax.fori_loop(0, 32, row, accs)

            neg = jnp.full((16,), -jnp.inf, _f32)
            nch = (end - abase + 31) // 32
            a0, a1, a2, a3 = lax.fori_loop(0, nch, ch_body,
                                           (neg, neg, neg, neg))
            obuf[gi, pl.ds(0, 16)] = a0
            obuf[gi, pl.ds(16, 16)] = a1
            obuf[gi, pl.ds(32, 16)] = a2
            obuf[gi, pl.ds(48, 16)] = a3
            return 0
        lax.fori_loop(0, 4, per_graph, 0)
        pltpu.sync_copy(obuf, oout.at[wid])


def _pool(x4, x1, x2, x3, pg0, pg1):
    k = pl.kernel(
        _pool_body,
        out_type=tuple(jax.ShapeDtypeStruct((32, 4, H), _f32)
                       for _ in range(4)),
        mesh=_mesh,
        scratch_types=[
            pltpu.VMEM((G_ACC,), _f32),
            pltpu.VMEM((G_ACC,), _f32),
            pltpu.VMEM((G + 16,), _i32),
            pltpu.VMEM((G + 16,), _i32),
            pltpu.VMEM((32, H), _f32),
            pltpu.VMEM((4, H), _f32),
            pltpu.SemaphoreType.DMA,
        ],
        compiler_params=_sc_params,
    )
    return k(x4, x1, x2, x3, pg0, pg1)


def _stage0_kernel(x_ref, p0_ref, p1_ref, wcat_ref, wc_ref, b_ref,
                   s0_ref, dis_ref):
    x = x_ref[...]
    iota = lax.broadcasted_iota(_i32, (R, CAT_TOT), 1)
    oh = jnp.zeros((R, CAT_TOT), _f32)
    for i, off in enumerate(CAT_OFFS):
        idx = x[:, i:i + 1].astype(_i32) + off
        oh = oh + (iota == idx).astype(_f32)
    lin = (jnp.dot(oh, wcat_ref[...], preferred_element_type=_f32)
           + jnp.dot(x[:, 8:16], wc_ref[...], preferred_element_type=_f32)
           + b_ref[...])
    dis = lax.rsqrt(1.0 + p0_ref[...] + p1_ref[...])
    dis_ref[...] = dis
    s0_ref[...] = lin * dis


def _stage0(x_p, p0, p1, wcat, wc, b):
    return pl.pallas_call(
        _stage0_kernel,
        grid=(N_PAD // R,),
        in_specs=[pl.BlockSpec((R, 16), lambda i: (i, 0)),
                  pl.BlockSpec((R, 1), lambda i: (i, 0)),
                  pl.BlockSpec((R, 1), lambda i: (i, 0)),
                  pl.BlockSpec((CAT_TOT, H), lambda i: (0, 0)),
                  pl.BlockSpec((8, H), lambda i: (0, 0)),
                  pl.BlockSpec((1, H), lambda i: (0, 0))],
        out_specs=[pl.BlockSpec((R, H), lambda i: (i, 0)),
                   pl.BlockSpec((R, 1), lambda i: (i, 0))],
        out_shape=[jax.ShapeDtypeStruct((N_PAD, H), _f32),
                   jax.ShapeDtypeStruct((N_PAD, 1), _f32)],
    )(x_p, p0, p1, wcat, wc, b)


def _stage_mid_kernel(agg_ref, s_ref, dis_ref, w_ref, b_ref, h_ref, sn_ref):
    d = dis_ref[...]
    hh = d * (agg_ref[...] + s_ref[...])
    hh = jnp.where(hh > 0, hh, ALPHA * hh)
    h_ref[...] = hh
    sn_ref[...] = (jnp.dot(hh, w_ref[...], preferred_element_type=_f32)
                   + b_ref[...]) * d


def _stage_mid(agg, s, dis, w, b):
    return pl.pallas_call(
        _stage_mid_kernel,
        grid=(N_PAD // R,),
        in_specs=[pl.BlockSpec((R, H), lambda i: (i, 0)),
                  pl.BlockSpec((R, H), lambda i: (i, 0)),
                  pl.BlockSpec((R, 1), lambda i: (i, 0)),
                  pl.BlockSpec((H, H), lambda i: (0, 0)),
                  pl.BlockSpec((1, H), lambda i: (0, 0))],
        out_specs=[pl.BlockSpec((R, H), lambda i: (i, 0)),
                   pl.BlockSpec((R, H), lambda i: (i, 0))],
        out_shape=[jax.ShapeDtypeStruct((N_PAD, H), _f32),
                   jax.ShapeDtypeStruct((N_PAD, H), _f32)],
    )(agg, s, dis, w, b)


def _stage3_kernel(agg_ref, s_ref, dis_ref, h0_ref, w_ref, b_ref,
                   x2_ref, x3_ref, sn_ref):
    d = dis_ref[...]
    hh = d * (agg_ref[...] + s_ref[...])
    hh = jnp.where(hh > 0, hh, ALPHA * hh)
    x2_ref[...] = hh
    x3 = h0_ref[...] + hh
    x3_ref[...] = x3
    sn_ref[...] = (jnp.dot(x3, w_ref[...], preferred_element_type=_f32)
                   + b_ref[...]) * d


def _stage3(agg, s, dis, h0, w, b):
    return pl.pallas_call(
        _stage3_kernel,
        grid=(N_PAD // R,),
        in_specs=[pl.BlockSpec((R, H), lambda i: (i, 0)),
                  pl.BlockSpec((R, H), lambda i: (i, 0)),
                  pl.BlockSpec((R, 1), lambda i: (i, 0)),
                  pl.BlockSpec((R, H), lambda i: (i, 0)),
                  pl.BlockSpec((H, H), lambda i: (0, 0)),
                  pl.BlockSpec((1, H), lambda i: (0, 0))],
        out_specs=[pl.BlockSpec((R, H), lambda i: (i, 0)),
                   pl.BlockSpec((R, H), lambda i: (i, 0)),
                   pl.BlockSpec((R, H), lambda i: (i, 0))],
        out_shape=[jax.ShapeDtypeStruct((N_PAD, H), _f32)] * 3,
    )(agg, s, dis, h0, w, b)


def _stage4_kernel(agg_ref, s_ref, dis_ref, x4_ref):
    hh = dis_ref[...] * (agg_ref[...] + s_ref[...])
    x4_ref[...] = jnp.where(hh > 0, hh, ALPHA * hh)


def _stage4(agg, s, dis):
    return pl.pallas_call(
        _stage4_kernel,
        grid=(N_PAD // R,),
        in_specs=[pl.BlockSpec((R, H), lambda i: (i, 0)),
                  pl.BlockSpec((R, H), lambda i: (i, 0)),
                  pl.BlockSpec((R, 1), lambda i: (i, 0))],
        out_specs=pl.BlockSpec((R, H), lambda i: (i, 0)),
        out_shape=jax.ShapeDtypeStruct((N_PAD, H), _f32),
    )(agg, s, dis)


def _mlp_kernel(p4_ref, p1_ref, p2_ref, p3_ref, wa_ref, wb_ref, wc_ref,
                wd_ref, b1_ref, gm_ref, bt_ref, w2_ref, b2_ref, o_ref):
    y = (jnp.dot(p4_ref[...], wa_ref[...], preferred_element_type=_f32)
         + jnp.dot(p1_ref[...], wb_ref[...], preferred_element_type=_f32)
         + jnp.dot(p2_ref[...], wc_ref[...], preferred_element_type=_f32)
         + jnp.dot(p3_ref[...], wd_ref[...], preferred_element_type=_f32)
         + b1_ref[...])
    mu = jnp.mean(y, axis=0, keepdims=True)
    var = jnp.mean((y - mu) * (y - mu), axis=0, keepdims=True)
    yn = (y - mu) * lax.rsqrt(var + 1e-5) * gm_ref[...] + bt_ref[...]
    yr = jnp.maximum(yn, 0.0)
    o_ref[...] = jnp.dot(yr, w2_ref[...], preferred_element_type=_f32) + b2_ref[...]


def _mlp(p4, p1, p2, p3, wa, wb, wc, wd, b1, gm, bt, w2, b2):
    return pl.pallas_call(
        _mlp_kernel,
        out_shape=jax.ShapeDtypeStruct((G, 1), _f32),
    )(p4, p1, p2, p3, wa, wb, wc, wd, b1, gm, bt, w2, b2)


def kernel(x, edge_index, batch, params):
    p = params
    src = edge_index[0].astype(_i32)
    dst = edge_index[1].astype(_i32)
    pad_e = E_PAD - E
    src_p = jnp.concatenate([src, jnp.zeros((pad_e,), _i32)])
    dst_msg = jnp.concatenate([dst, jnp.full((pad_e,), 1 << 30, _i32)])
    dst_deg = jnp.concatenate([dst, jnp.full((pad_e,), N_PAD - 1, _i32)])
    batch_p = jnp.concatenate([batch.astype(_i32),
                               jnp.full((N_PAD - N,), G_ACC - 1, _i32)])
    x_p = jnp.pad(x, ((0, N_PAD - N), (0, 0)))

    pd0, pd1, pg0, pg1 = _stats(dst_deg, batch_p)
    p0 = pd0.reshape(N_PAD, 1)
    p1 = pd1.reshape(N_PAD, 1)

    bsrc, bdst, bcnt = _bucket(src_p, dst_msg)

    wcat = jnp.pad(p['W_init'][:31], ((0, 1), (0, 0)))
    wc = p['W_init'][31:]
    s0, dis = _stage0(x_p, p0, p1, wcat, wc, p['b_init'].reshape(1, H))

    agg0 = _msgpass(s0, bsrc, bdst, bcnt)
    h0, s1 = _stage_mid(agg0, s0, dis, p['W_head'], p['b_head'].reshape(1, H))
    agg1 = _msgpass(s1, bsrc, bdst, bcnt)
    x1, s2 = _stage_mid(agg1, s1, dis, p['W_body'], p['b_body'].reshape(1, H))
    agg2 = _msgpass(s2, bsrc, bdst, bcnt)
    x2, x3, s3 = _stage3(agg2, s2, dis, h0, p['W_tail'],
                         p['b_tail'].reshape(1, H))
    agg3 = _msgpass(s3, bsrc, bdst, bcnt)
    x4 = _stage4(agg3, s3, dis)

    p4, p1m, p2m, p3m = _pool(x4, x1, x2, x3, pg0, pg1)
    p4, p1m, p2m, p3m = (a.reshape(G, H) for a in (p4, p1m, p2m, p3m))

    w1 = p['W_mlp1']
    out = _mlp(p4, p1m, p2m, p3m,
               w1[0:64], w1[64:128], w1[128:192], w1[192:256],
               p['b_mlp1'].reshape(1, H),
               p['bn_gamma'].reshape(1, H), p['bn_beta'].reshape(1, H),
               p['W_mlp2'], p['b_mlp2'].reshape(1, 1))
    return out.reshape(G)


# speedup vs baseline: 1.2050x; 1.0054x over previous
import jax
import jax.numpy as jnp
from jax import lax
from jax.experimental import pallas as pl
from jax.experimental.pallas import tpu as pltpu
from jax.experimental.pallas import tpu_sc as plsc

N = 50000
E = 800000
G = 128
H = 64
ALPHA = 0.01

N_PAD = 50176
E_PAD = 802816
HALF = N_PAD // 2
ACC_ROWS = HALF + 16
CHUNK = 128
MSTAGE = 3584
NCH_ACC = HALF // CHUNK
G_ACC = 256
R = 512
CAT_OFFS = (0, 7, 12, 16, 20, 22, 24, 28)
CAT_TOT = 32

_mesh = plsc.VectorSubcoreMesh(core_axis_name="c", subcore_axis_name="s")
_f32 = jnp.float32
_i32 = jnp.int32
_sc_params = pltpu.CompilerParams(needs_layout_passes=False,
                                  use_tc_tiling_on_sc=False)


NZCH = N_PAD // 512


def _stats_body(dst_hbm, batch_hbm, pd0_hbm, pd1_hbm, pg0_hbm, pg1_hbm,
                estage, bstage, dchunk, bchunk, ones, zbuf, accd, accg, sem):
    cid = lax.axis_index("c")
    sid = lax.axis_index("s")
    wid = sid * 2 + cid

    def zb(i, _):
        zbuf[pl.ds(i * 16, 16)] = jnp.zeros((16,), _f32)
        return 0
    lax.fori_loop(0, 32, zb, 0)
    for j in range(CHUNK // 16):
        ones[pl.ds(j * 16, 16)] = jnp.ones((16,), _f32)

    def za(k, _):
        c = sid + k * 16
        @pl.when(c < NZCH)
        def _():
            pltpu.sync_copy(zbuf, accd.at[pl.ds(c * 512, 512)])
        return 0
    lax.fori_loop(0, 7, za, 0)

    @pl.when(sid == 0)
    def _():
        pltpu.sync_copy(zbuf.at[pl.ds(0, G_ACC)], accg)
    plsc.subcore_barrier()

    pltpu.sync_copy(dst_hbm.at[pl.ds(wid * (E_PAD // 32), E_PAD // 32)], estage)

    def deg_chunk(i, _):
        def cp(j, _):
            dchunk[pl.ds(j * 16, 16)] = estage[pl.ds(i * CHUNK + j * 16, 16)]
            return 0
        lax.fori_loop(0, CHUNK // 16, cp, 0)
        pltpu.sync_copy(ones, accd.at[dchunk], add=True)
        return 0
    lax.fori_loop(0, (E_PAD // 32) // CHUNK, deg_chunk, 0)

    def g_outer(k, _):
        c = wid + k * 32
        @pl.when(c < NZCH)
        def _():
            pltpu.sync_copy(batch_hbm.at[pl.ds(c * 512, 512)], bstage)
            def g_chunk(i, _):
                bchunk[pl.ds(0, 16)] = bstage[pl.ds(i * 16, 16)]
                pltpu.sync_copy(ones.at[pl.ds(0, 16)], accg.at[bchunk],
                                add=True)
                return 0
            lax.fori_loop(0, 32, g_chunk, 0)
        return 0
    lax.fori_loop(0, 4, g_outer, 0)

    plsc.subcore_barrier()
    def co(k, _):
        c = sid + k * 16
        @pl.when((c < NZCH) & (cid == 0))
        def _():
            pltpu.sync_copy(accd.at[pl.ds(c * 512, 512)],
                            pd0_hbm.at[pl.ds(c * 512, 512)])
        @pl.when((c < NZCH) & (cid == 1))
        def _():
            pltpu.sync_copy(accd.at[pl.ds(c * 512, 512)],
                            pd1_hbm.at[pl.ds(c * 512, 512)])
        return 0
    lax.fori_loop(0, 7, co, 0)

    @pl.when((sid == 0) & (cid == 0))
    def _():
        pltpu.sync_copy(accg, pg0_hbm)

    @pl.when((sid == 0) & (cid == 1))
    def _():
        pltpu.sync_copy(accg, pg1_hbm)


def _stats(dst_deg, batch_p):
    k = pl.kernel(
        _stats_body,
        out_type=(jax.ShapeDtypeStruct((N_PAD,), _f32),
                  jax.ShapeDtypeStruct((N_PAD,), _f32),
                  jax.ShapeDtypeStruct((G_ACC,), _f32),
                  jax.ShapeDtypeStruct((G_ACC,), _f32)),
        mesh=_mesh,
        scratch_types=[
            pltpu.VMEM((E_PAD // 32,), _i32),
            pltpu.VMEM((512,), _i32),
            pltpu.VMEM((CHUNK,), _i32),
            pltpu.VMEM((16,), _i32),
            pltpu.VMEM((CHUNK,), _f32),
            pltpu.VMEM((512,), _f32),
            pltpu.VMEM_SHARED((N_PAD,), _f32),
            pltpu.VMEM_SHARED((G_ACC,), _f32),
            pltpu.SemaphoreType.DMA,
        ],
    )
    return k(dst_deg, batch_p)


CAP = 32256
BLK = 16 * CHUNK
BUFSZ = CAP + MSTAGE + BLK + 32
TRASH = CAP + MSTAGE + BLK


def _bucket_body(src_hbm, dst_hbm, bsrc_hbm, bdst_hbm, bcnt_hbm,
                 srcbuf, dstbuf, sstage, dstage, cbuf, sem):
    cid = lax.axis_index("c")
    sid = lax.axis_index("s")
    lo = cid * HALF
    ebase = sid * (E_PAD // 16)

    def outer(c, ptr):
        pltpu.sync_copy(src_hbm.at[pl.ds(ebase + c * MSTAGE, MSTAGE)], sstage)
        pltpu.sync_copy(dst_hbm.at[pl.ds(ebase + c * MSTAGE, MSTAGE)], dstage)

        def inner(i, ptr):
            s16 = sstage[pl.ds(i * 16, 16)]
            d16 = dstage[pl.ds(i * 16, 16)]
            m = (d16 >= lo) & (d16 < lo + HALF)
            mi = m.astype(_i32)
            cum = plsc.cumsum(mi)
            lane = lax.iota(_i32, 16)
            pos = jnp.where(m, ptr + cum - mi, TRASH + lane)
            plsc.store_scatter(srcbuf, [pos], s16)
            plsc.store_scatter(dstbuf, [pos], d16 - lo)
            return ptr + jnp.sum(mi)
        return lax.fori_loop(0, MSTAGE // 16, inner, ptr)
    kcnt = lax.fori_loop(0, (E_PAD // 16) // MSTAGE, outer,
                         jnp.asarray(0, _i32))

    def fill(j, _):
        srcbuf[pl.ds(kcnt + j * 16, 16)] = jnp.zeros((16,), _i32)
        dstbuf[pl.ds(kcnt + j * 16, 16)] = HALF + lax.iota(_i32, 16)
        return 0
    lax.fori_loop(0, (MSTAGE + 256) // 16, fill, 0)

    base = (cid * 16 + sid) * CAP
    nblk = (kcnt + 256 + MSTAGE - 1) // MSTAGE

    def wr(c, _):
        pltpu.sync_copy(srcbuf.at[pl.ds(c * MSTAGE, MSTAGE)],
                        bsrc_hbm.at[pl.ds(base + c * MSTAGE, MSTAGE)])
        pltpu.sync_copy(dstbuf.at[pl.ds(c * MSTAGE, MSTAGE)],
                        bdst_hbm.at[pl.ds(base + c * MSTAGE, MSTAGE)])
        return 0
    lax.fori_loop(0, nblk, wr, 0)

    nch = ((kcnt + 2 * CHUNK - 1) // (2 * CHUNK)) * 2
    cbuf[pl.ds(0, 16)] = jnp.zeros((16,), _i32) + nch
    pltpu.sync_copy(cbuf, bcnt_hbm.at[pl.ds((cid * 16 + sid) * 128, 128)])


def _bucket(src_p, dst_p):
    k = pl.kernel(
        _bucket_body,
        out_type=(jax.ShapeDtypeStruct((32 * CAP,), _i32),
                  jax.ShapeDtypeStruct((32 * CAP,), _i32),
                  jax.ShapeDtypeStruct((32 * 128,), _i32)),
        mesh=_mesh,
        scratch_types=[
            pltpu.VMEM((BUFSZ,), _i32),
            pltpu.VMEM((BUFSZ,), _i32),
            pltpu.VMEM((MSTAGE,), _i32),
            pltpu.VMEM((MSTAGE,), _i32),
            pltpu.VMEM((128,), _i32),
            pltpu.SemaphoreType.DMA,
        ],
        compiler_params=_sc_params,
    )
    return k(src_p, dst_p)


def _msgpass_body(s_hbm, bsrc_hbm, bdst_hbm, bcnt_hbm, out_hbm,
                  rows0, rows1, sblk, dblk, dch0, dch1, cbuf, acc,
                  semi, semg, semg1, sems):
    cid = lax.axis_index("c")
    sid = lax.axis_index("s")
    lo = cid * HALF

    def zr(i, _):
        rows0[i // 4, pl.ds((i % 4) * 16, 16)] = jnp.zeros((16,), _f32)
        return 0
    lax.fori_loop(0, CHUNK * 4, zr, 0)

    def za(k, _):
        c = sid + k * 16
        @pl.when(c < NCH_ACC)
        def _():
            pltpu.sync_copy(rows0, acc.at[pl.ds(c * CHUNK, CHUNK)])
        return 0
    lax.fori_loop(0, 13, za, 0)
    plsc.subcore_barrier()

    pltpu.sync_copy(bcnt_hbm.at[pl.ds((cid * 16 + sid) * 128, 128)], cbuf)
    nch = cbuf[pl.ds(0, 16)][0]
    base = (cid * 16 + sid) * CAP

    def chunk_body(c2, _):
        c0 = 2 * c2
        i0s = pltpu.async_copy(bsrc_hbm.at[pl.ds(base + c0 * CHUNK, CHUNK)],
                               sblk, semi)
        i0d = pltpu.async_copy(bdst_hbm.at[pl.ds(base + c0 * CHUNK, CHUNK)],
                               dch0, semi)
        i1s = pltpu.async_copy(
            bsrc_hbm.at[pl.ds(base + (c0 + 1) * CHUNK, CHUNK)], dblk, semi)
        i1d = pltpu.async_copy(
            bdst_hbm.at[pl.ds(base + (c0 + 1) * CHUNK, CHUNK)], dch1, semi)
        i0s.wait()
        i0d.wait()
        i1s.wait()
        i1d.wait()
        g0 = pltpu.async_copy(s_hbm.at[sblk], rows0, semg)
        g1 = pltpu.async_copy(s_hbm.at[dblk], rows1, semg1)
        g0.wait()
        s0 = pltpu.async_copy(rows0, acc.at[dch0], sems, add=True)
        g1.wait()
        s1 = pltpu.async_copy(rows1, acc.at[dch1], sems, add=True)
        s0.wait()
        s1.wait()
        return 0
    lax.fori_loop(0, nch // 2, chunk_body, 0)

    plsc.subcore_barrier()
    def co(k, _):
        c = sid + k * 16
        @pl.when(c < NCH_ACC)
        def _():
            pltpu.sync_copy(acc.at[pl.ds(c * CHUNK, CHUNK)],
                            out_hbm.at[pl.ds(lo + c * CHUNK, CHUNK)])
        return 0
    lax.fori_loop(0, 13, co, 0)


def _msgpass(s, bsrc, bdst, bcnt):
    k = pl.kernel(
        _msgpass_body,
        out_type=jax.ShapeDtypeStruct((N_PAD, H), _f32),
        mesh=_mesh,
        scratch_types=[
            pltpu.VMEM((CHUNK, H), _f32),
            pltpu.VMEM((CHUNK, H), _f32),
            pltpu.VMEM((CHUNK,), _i32),
            pltpu.VMEM((CHUNK,), _i32),
            pltpu.VMEM((CHUNK,), _i32),
            pltpu.VMEM((CHUNK,), _i32),
            pltpu.VMEM((128,), _i32),
            pltpu.VMEM_SHARED((ACC_ROWS, H), _f32),
            pltpu.SemaphoreType.DMA,
            pltpu.SemaphoreType.DMA,
            pltpu.SemaphoreType.DMA,
            pltpu.SemaphoreType.DMA,
        ],
        compiler_params=_sc_params,
    )
    return k(s, bsrc, bdst, bcnt)


def _pool_body(x4, x1, x2, x3, st_hbm, cn_hbm, o4, o1, o2, o3,
               st_s, cn_s, buf, obuf, sem):
    cid = lax.axis_index("c")
    sid = lax.axis_index("s")
    wid = sid * 2 + cid
    pltpu.sync_copy(st_hbm, st_s.at[pl.ds(0, G)])
    pltpu.sync_copy(cn_hbm, cn_s.at[pl.ds(0, G)])

    for xin, oout in ((x4, o4), (x1, o1), (x2, o2), (x3, o3)):
        def per_graph(gi, _, xin=xin):
            g = wid * 4 + gi
            start = st_s[pl.ds(g, 16)][0]
            end = start + cn_s[pl.ds(g, 16)][0]
            abase = (start // 8) * 8

            def ch_body(c, accs):
                pltpu.sync_copy(xin.at[pl.ds(abase + c * 32, 32)], buf)

                def row(rr, accs):
                    a0, a1, a2, a3 = accs
                    r = abase + c * 32 + rr
                    pred = (r >= start) & (r < end)
                    v0 = buf[rr, pl.ds(0, 16)]
                    v1 = buf[rr, pl.ds(16, 16)]
                    v2 = buf[rr, pl.ds(32, 16)]
                    v3 = buf[rr, pl.ds(48, 16)]
                    a0 = jnp.where(pred, jnp.maximum(a0, v0), a0)
                    a1 = jnp.where(pred, jnp.maximum(a1, v1), a1)
                    a2 = jnp.where(pred, jnp.maximum(a2, v2), a2)
                    a3 = jnp.where(pred, jnp.maximum(a3, v3), a3)
                    return (a0, a1, a2, a3)
                return lax.fori_loop(0, 32, row, accs)

            neg = jnp.full((16,), -jnp.inf, _f32)
            nch = (end - abase + 31) // 32
            a0, a1, a2, a3 = lax.fori_loop(0, nch, ch_body,
                                           (neg, neg, neg, neg))
            obuf[gi, pl.ds(0, 16)] = a0
            obuf[gi, pl.ds(16, 16)] = a1
            obuf[gi, pl.ds(32, 16)] = a2
            obuf[gi, pl.ds(48, 16)] = a3
            return 0
        lax.fori_loop(0, 4, per_graph, 0)
        pltpu.sync_copy(obuf, oout.at[wid])


def _pool(x4, x1, x2, x3, starts, cnts):
    k = pl.kernel(
        _pool_body,
        out_type=tuple(jax.ShapeDtypeStruct((32, 4, H), _f32)
                       for _ in range(4)),
        mesh=_mesh,
        scratch_types=[
            pltpu.VMEM((G + 16,), _i32),
            pltpu.VMEM((G + 16,), _i32),
            pltpu.VMEM((32, H), _f32),
            pltpu.VMEM((4, H), _f32),
            pltpu.SemaphoreType.DMA,
        ],
    )
    return k(x4, x1, x2, x3, starts, cnts)


def _stage0_kernel(x_ref, p0_ref, p1_ref, wcat_ref, wc_ref, b_ref,
                   s0_ref, dis_ref):
    x = x_ref[...]
    iota = lax.broadcasted_iota(_i32, (R, CAT_TOT), 1)
    oh = jnp.zeros((R, CAT_TOT), _f32)
    for i, off in enumerate(CAT_OFFS):
        idx = x[:, i:i + 1].astype(_i32) + off
        oh = oh + (iota == idx).astype(_f32)
    lin = (jnp.dot(oh, wcat_ref[...], preferred_element_type=_f32)
           + jnp.dot(x[:, 8:16], wc_ref[...], preferred_element_type=_f32)
           + b_ref[...])
    dis = lax.rsqrt(1.0 + p0_ref[...] + p1_ref[...])
    dis_ref[...] = dis
    s0_ref[...] = lin * dis


def _stage0(x_p, p0, p1, wcat, wc, b):
    return pl.pallas_call(
        _stage0_kernel,
        grid=(N_PAD // R,),
        in_specs=[pl.BlockSpec((R, 16), lambda i: (i, 0)),
                  pl.BlockSpec((R, 1), lambda i: (i, 0)),
                  pl.BlockSpec((R, 1), lambda i: (i, 0)),
                  pl.BlockSpec((CAT_TOT, H), lambda i: (0, 0)),
                  pl.BlockSpec((8, H), lambda i: (0, 0)),
                  pl.BlockSpec((1, H), lambda i: (0, 0))],
        out_specs=[pl.BlockSpec((R, H), lambda i: (i, 0)),
                   pl.BlockSpec((R, 1), lambda i: (i, 0))],
        out_shape=[jax.ShapeDtypeStruct((N_PAD, H), _f32),
                   jax.ShapeDtypeStruct((N_PAD, 1), _f32)],
    )(x_p, p0, p1, wcat, wc, b)


def _stage_mid_kernel(agg_ref, s_ref, dis_ref, w_ref, b_ref, h_ref, sn_ref):
    d = dis_ref[...]
    hh = d * (agg_ref[...] + s_ref[...])
    hh = jnp.where(hh > 0, hh, ALPHA * hh)
    h_ref[...] = hh
    sn_ref[...] = (jnp.dot(hh, w_ref[...], preferred_element_type=_f32)
                   + b_ref[...]) * d


def _stage_mid(agg, s, dis, w, b):
    return pl.pallas_call(
        _stage_mid_kernel,
        grid=(N_PAD // R,),
        in_specs=[pl.BlockSpec((R, H), lambda i: (i, 0)),
                  pl.BlockSpec((R, H), lambda i: (i, 0)),
                  pl.BlockSpec((R, 1), lambda i: (i, 0)),
                  pl.BlockSpec((H, H), lambda i: (0, 0)),
                  pl.BlockSpec((1, H), lambda i: (0, 0))],
        out_specs=[pl.BlockSpec((R, H), lambda i: (i, 0)),
                   pl.BlockSpec((R, H), lambda i: (i, 0))],
        out_shape=[jax.ShapeDtypeStruct((N_PAD, H), _f32),
                   jax.ShapeDtypeStruct((N_PAD, H), _f32)],
    )(agg, s, dis, w, b)


def _stage3_kernel(agg_ref, s_ref, dis_ref, h0_ref, w_ref, b_ref,
                   x2_ref, x3_ref, sn_ref):
    d = dis_ref[...]
    hh = d * (agg_ref[...] + s_ref[...])
    hh = jnp.where(hh > 0, hh, ALPHA * hh)
    x2_ref[...] = hh
    x3 = h0_ref[...] + hh
    x3_ref[...] = x3
    sn_ref[...] = (jnp.dot(x3, w_ref[...], preferred_element_type=_f32)
                   + b_ref[...]) * d


def _stage3(agg, s, dis, h0, w, b):
    return pl.pallas_call(
        _stage3_kernel,
        grid=(N_PAD // R,),
        in_specs=[pl.BlockSpec((R, H), lambda i: (i, 0)),
                  pl.BlockSpec((R, H), lambda i: (i, 0)),
                  pl.BlockSpec((R, 1), lambda i: (i, 0)),
                  pl.BlockSpec((R, H), lambda i: (i, 0)),
                  pl.BlockSpec((H, H), lambda i: (0, 0)),
                  pl.BlockSpec((1, H), lambda i: (0, 0))],
        out_specs=[pl.BlockSpec((R, H), lambda i: (i, 0)),
                   pl.BlockSpec((R, H), lambda i: (i, 0)),
                   pl.BlockSpec((R, H), lambda i: (i, 0))],
        out_shape=[jax.ShapeDtypeStruct((N_PAD, H), _f32)] * 3,
    )(agg, s, dis, h0, w, b)


def _stage4_kernel(agg_ref, s_ref, dis_ref, x4_ref):
    hh = dis_ref[...] * (agg_ref[...] + s_ref[...])
    x4_ref[...] = jnp.where(hh > 0, hh, ALPHA * hh)


def _stage4(agg, s, dis):
    return pl.pallas_call(
        _stage4_kernel,
        grid=(N_PAD // R,),
        in_specs=[pl.BlockSpec((R, H), lambda i: (i, 0)),
                  pl.BlockSpec((R, H), lambda i: (i, 0)),
                  pl.BlockSpec((R, 1), lambda i: (i, 0))],
        out_specs=pl.BlockSpec((R, H), lambda i: (i, 0)),
        out_shape=jax.ShapeDtypeStruct((N_PAD, H), _f32),
    )(agg, s, dis)


def _starts_kernel(pg0_ref, pg1_ref, st_ref, cn_ref):
    cnt = pg0_ref[...] + pg1_ref[...]
    r = lax.broadcasted_iota(_i32, (G_ACC, G_ACC), 0)
    c = lax.broadcasted_iota(_i32, (G_ACC, G_ACC), 1)
    m = ((r < c) & (r < G)).astype(_f32)
    st = jnp.dot(cnt, m, preferred_element_type=_f32,
                 precision=lax.Precision.HIGHEST)
    st_ref[...] = st.astype(_i32)
    cn_ref[...] = cnt.astype(_i32)


def _starts(pg0, pg1):
    return pl.pallas_call(
        _starts_kernel,
        out_shape=[jax.ShapeDtypeStruct((1, G_ACC), _i32),
                   jax.ShapeDtypeStruct((1, G_ACC), _i32)],
    )(pg0, pg1)


def _mlp_kernel(p4_ref, p1_ref, p2_ref, p3_ref, wa_ref, wb_ref, wc_ref,
                wd_ref, b1_ref, gm_ref, bt_ref, w2_ref, b2_ref, o_ref):
    y = (jnp.dot(p4_ref[...], wa_ref[...], preferred_element_type=_f32)
         + jnp.dot(p1_ref[...], wb_ref[...], preferred_element_type=_f32)
         + jnp.dot(p2_ref[...], wc_ref[...], preferred_element_type=_f32)
         + jnp.dot(p3_ref[...], wd_ref[...], preferred_element_type=_f32)
         + b1_ref[...])
    mu = jnp.mean(y, axis=0, keepdims=True)
    var = jnp.mean((y - mu) * (y - mu), axis=0, keepdims=True)
    yn = (y - mu) * lax.rsqrt(var + 1e-5) * gm_ref[...] + bt_ref[...]
    yr = jnp.maximum(yn, 0.0)
    o_ref[...] = jnp.dot(yr, w2_ref[...], preferred_element_type=_f32) + b2_ref[...]


def _mlp(p4, p1, p2, p3, wa, wb, wc, wd, b1, gm, bt, w2, b2):
    return pl.pallas_call(
        _mlp_kernel,
        out_shape=jax.ShapeDtypeStruct((G, 1), _f32),
    )(p4, p1, p2, p3, wa, wb, wc, wd, b1, gm, bt, w2, b2)


def kernel(x, edge_index, batch, params):
    p = params
    src = edge_index[0].astype(_i32)
    dst = edge_index[1].astype(_i32)
    pad_e = E_PAD - E
    src_p = jnp.concatenate([src, jnp.zeros((pad_e,), _i32)])
    dst_msg = jnp.concatenate([dst, jnp.full((pad_e,), 1 << 30, _i32)])
    dst_deg = jnp.concatenate([dst, jnp.full((pad_e,), N_PAD - 1, _i32)])
    batch_p = jnp.concatenate([batch.astype(_i32),
                               jnp.full((N_PAD - N,), G_ACC - 1, _i32)])
    x_p = jnp.pad(x, ((0, N_PAD - N), (0, 0)))

    pd0, pd1, pg0, pg1 = _stats(dst_deg, batch_p)
    p0 = pd0.reshape(N_PAD, 1)
    p1 = pd1.reshape(N_PAD, 1)

    bsrc, bdst, bcnt = _bucket(src_p, dst_msg)

    wcat = jnp.pad(p['W_init'][:31], ((0, 1), (0, 0)))
    wc = p['W_init'][31:]
    s0, dis = _stage0(x_p, p0, p1, wcat, wc, p['b_init'].reshape(1, H))

    agg0 = _msgpass(s0, bsrc, bdst, bcnt)
    h0, s1 = _stage_mid(agg0, s0, dis, p['W_head'], p['b_head'].reshape(1, H))
    agg1 = _msgpass(s1, bsrc, bdst, bcnt)
    x1, s2 = _stage_mid(agg1, s1, dis, p['W_body'], p['b_body'].reshape(1, H))
    agg2 = _msgpass(s2, bsrc, bdst, bcnt)
    x2, x3, s3 = _stage3(agg2, s2, dis, h0, p['W_tail'],
                         p['b_tail'].reshape(1, H))
    agg3 = _msgpass(s3, bsrc, bdst, bcnt)
    x4 = _stage4(agg3, s3, dis)

    st_i, cn_i = _starts(pg0.reshape(1, G_ACC), pg1.reshape(1, G_ACC))
    starts = st_i[0, :G]
    cnts = cn_i[0, :G]
    p4, p1m, p2m, p3m = _pool(x4, x1, x2, x3, starts, cnts)
    p4, p1m, p2m, p3m = (a.reshape(G, H) for a in (p4, p1m, p2m, p3m))

    w1 = p['W_mlp1']
    out = _mlp(p4, p1m, p2m, p3m,
               w1[0:64], w1[64:128], w1[128:192], w1[192:256],
               p['b_mlp1'].reshape(1, H),
               p['bn_gamma'].reshape(1, H), p['bn_beta'].reshape(1, H),
               p['W_mlp2'], p['b_mlp2'].reshape(1, 1))
    return out.reshape(G)
